# baseline (speedup 1.0000x reference)
import jax
import jax.numpy as jnp
from jax import lax
from jax.experimental import pallas as pl
from jax.experimental.pallas import tpu as pltpu
from jax.experimental.pallas import tpu_sc as plsc

BATCH = 16384
D = 32
NC = 2
NS = 16
LANES = 16
NW = NC * NS
BPW = BATCH // NW
CHUNK = 128
NCHUNK = BPW // CHUNK
NR = 8
NGRP = BPW // LANES


def _mf_body(uid_hbm, iid_hbm, pt_hbm, qt_hbm, ub_hbm, ib_hbm, out_hbm,
             uc_v, ic_v, uring, iring, ubch, ibch, out_v, sem, sem2, sem3):
    wid = lax.axis_index("s") * NC + lax.axis_index("c")

    pltpu.sync_copy(uid_hbm.at[wid], uc_v)
    pltpu.sync_copy(iid_hbm.at[wid], ic_v)

    bcopies = []
    for k in range(NCHUNK):
        bcopies.append(pltpu.async_copy(ub_hbm.at[uc_v.at[k]], ubch.at[k], sem))
        bcopies.append(pltpu.async_copy(ib_hbm.at[ic_v.at[k]], ibch.at[k], sem))
    for c in bcopies:
        c.wait()

    iota = lax.iota(jnp.int32, LANES)
    cg_lo = lax.shift_right_logical(iota, 3)
    cg_hi = cg_lo + 2
    cl16 = jnp.bitwise_and(iota, 7)

    RW = 4
    RIDS = LANES // RW

    def fire_round(uv16, iv16, rw, sems):
        s = sems[rw % 3]
        fired = []
        for j in range(RIDS):
            slot = (rw % 3) * RIDS + j
            uid = uv16[rw * RIDS + j]
            iid = iv16[rw * RIDS + j]
            ut = lax.shift_right_logical(uid, 7) * 128
            it = lax.shift_right_logical(iid, 7) * 128
            fired.append(pltpu.async_copy(
                pt_hbm.at[:, :, pl.ds(ut, 128)], uring.at[slot], s))
            fired.append(pltpu.async_copy(
                qt_hbm.at[:, :, pl.ds(it, 128)], iring.at[slot], s))
        return fired

    def compute_round(uv16, iv16, rw, acc):
        for j in range(RIDS):
            slot = (rw % 3) * RIDS + j
            uid = uv16[rw * RIDS + j]
            iid = iv16[rw * RIDS + j]
            uil = jnp.broadcast_to(jnp.bitwise_and(uid, 127), (LANES,))
            iil = jnp.broadcast_to(jnp.bitwise_and(iid, 127), (LANES,))
            u_lo = plsc.load_gather(uring.at[slot], [cg_lo, cl16, uil])
            u_hi = plsc.load_gather(uring.at[slot], [cg_hi, cl16, uil])
            i_lo = plsc.load_gather(iring.at[slot], [cg_lo, cl16, iil])
            i_hi = plsc.load_gather(iring.at[slot], [cg_hi, cl16, iil])
            prod = u_lo * i_lo + u_hi * i_hi
            acc = jnp.where(iota == (rw * RIDS + j),
                            jnp.broadcast_to(jnp.sum(prod), (LANES,)), acc)
        return acc

    def group_body(g, carry):
        row = lax.shift_right_logical(g, 3)
        colb = jnp.bitwise_and(g, 7) * LANES
        sl = pl.ds(colb, LANES)
        uv16 = uc_v[row, sl]
        iv16 = ic_v[row, sl]
        acc = jnp.zeros((LANES,), jnp.float32)
        sems = (sem, sem2, sem3)

        pend = {0: fire_round(uv16, iv16, 0, sems),
                1: fire_round(uv16, iv16, 1, sems),
                2: fire_round(uv16, iv16, 2, sems)}
        for rw in range(RW):
            for c in pend.pop(rw):
                c.wait()
            if rw + 3 < RW:
                pend[rw + 3] = fire_round(uv16, iv16, rw + 3, sems)
            acc = compute_round(uv16, iv16, rw, acc)

        out_v[row, sl] = acc + ubch[row, sl] + ibch[row, sl]
        return carry

    lax.fori_loop(0, NGRP, group_body, 0)

    pltpu.sync_copy(out_v, out_hbm.at[wid])


@jax.jit
def _mf_sc(uidx, iidx, pt, qt, ub, ib):
    mesh = plsc.VectorSubcoreMesh(core_axis_name="c", subcore_axis_name="s")
    f = pl.kernel(
        _mf_body,
        out_type=jax.ShapeDtypeStruct((NW, NCHUNK, CHUNK), jnp.float32),
        mesh=mesh,
        compiler_params=pltpu.CompilerParams(
            needs_layout_passes=False, use_tc_tiling_on_sc=True),
        scratch_types=[
            pltpu.VMEM((NCHUNK, CHUNK), jnp.int32),
            pltpu.VMEM((NCHUNK, CHUNK), jnp.int32),
            pltpu.VMEM((12, 4, 8, 128), jnp.float32),
            pltpu.VMEM((12, 4, 8, 128), jnp.float32),
            pltpu.VMEM((NCHUNK, CHUNK), jnp.float32),
            pltpu.VMEM((NCHUNK, CHUNK), jnp.float32),
            pltpu.VMEM((NCHUNK, CHUNK), jnp.float32),
            pltpu.SemaphoreType.DMA,
            pltpu.SemaphoreType.DMA,
            pltpu.SemaphoreType.DMA,
        ],
    )
    return f(uidx, iidx, pt, qt, ub, ib)


def kernel(user_id, item_id, p, q, user_bias, item_bias):
    uidx = (user_id - 1).reshape(NW, NCHUNK, CHUNK)
    iidx = (item_id - 1).reshape(NW, NCHUNK, CHUNK)
    pt = jnp.transpose(p).reshape(4, 8, 1000000)
    qt = jnp.transpose(q).reshape(4, 8, 1000000)
    ub = jnp.sum(user_bias, axis=1)
    ib = jnp.sum(item_bias, axis=1)
    out = _mf_sc(uidx, iidx, pt, qt, ub, ib)
    return out.reshape(BATCH, 1)


# speedup vs baseline: 1.4538x; 1.4538x over previous
import jax
import jax.numpy as jnp
from jax import lax
from jax.experimental import pallas as pl
from jax.experimental.pallas import tpu as pltpu
from jax.experimental.pallas import tpu_sc as plsc

BATCH = 16384
D = 32
NC = 2
NS = 16
LANES = 16
NW = NC * NS
BPW = BATCH // NW
CHUNK = 128
NCHUNK = BPW // CHUNK
NR = 8
NGRP = BPW // LANES


def _mf_body(uid_hbm, iid_hbm, pt_hbm, qt_hbm, ub_hbm, ib_hbm, out_hbm,
             uc_v, ic_v, uring, iring, ubch, ibch, out_v, sem, sem2, sem3):
    wid = lax.axis_index("s") * NC + lax.axis_index("c")

    pltpu.sync_copy(uid_hbm.at[wid], uc_v)
    pltpu.sync_copy(iid_hbm.at[wid], ic_v)

    bcopies = []
    for k in range(NCHUNK):
        bcopies.append(pltpu.async_copy(ub_hbm.at[uc_v.at[k]], ubch.at[k], sem))
        bcopies.append(pltpu.async_copy(ib_hbm.at[ic_v.at[k]], ibch.at[k], sem))
    for c in bcopies:
        c.wait()

    iota = lax.iota(jnp.int32, LANES)
    cg_lo = lax.shift_right_logical(iota, 3)
    cg_hi = cg_lo + 2
    cl16 = jnp.bitwise_and(iota, 7)

    RW = 4
    RIDS = LANES // RW

    def fire_round(uv16, iv16, rw, sems):
        s = sems[rw % 3]
        fired = []
        for j in range(RIDS):
            slot = (rw % 3) * RIDS + j
            uid = uv16[rw * RIDS + j]
            iid = iv16[rw * RIDS + j]
            ut = lax.shift_right_logical(uid, 4) * 16
            it = lax.shift_right_logical(iid, 4) * 16
            fired.append(pltpu.async_copy(
                pt_hbm.at[:, :, pl.ds(ut, 16)],
                uring.at[slot, :, :, pl.ds(0, 16)], s))
            fired.append(pltpu.async_copy(
                qt_hbm.at[:, :, pl.ds(it, 16)],
                iring.at[slot, :, :, pl.ds(0, 16)], s))
        return fired

    def compute_round(uv16, iv16, rw, acc):
        for j in range(RIDS):
            slot = (rw % 3) * RIDS + j
            uid = uv16[rw * RIDS + j]
            iid = iv16[rw * RIDS + j]
            uil = jnp.broadcast_to(jnp.bitwise_and(uid, 15), (LANES,))
            iil = jnp.broadcast_to(jnp.bitwise_and(iid, 15), (LANES,))
            u_lo = plsc.load_gather(uring.at[slot], [cg_lo, cl16, uil])
            u_hi = plsc.load_gather(uring.at[slot], [cg_hi, cl16, uil])
            i_lo = plsc.load_gather(iring.at[slot], [cg_lo, cl16, iil])
            i_hi = plsc.load_gather(iring.at[slot], [cg_hi, cl16, iil])
            prod = u_lo * i_lo + u_hi * i_hi
            acc = jnp.where(iota == (rw * RIDS + j),
                            jnp.broadcast_to(jnp.sum(prod), (LANES,)), acc)
        return acc

    def group_body(g, carry):
        row = lax.shift_right_logical(g, 3)
        colb = jnp.bitwise_and(g, 7) * LANES
        sl = pl.ds(colb, LANES)
        uv16 = uc_v[row, sl]
        iv16 = ic_v[row, sl]
        acc = jnp.zeros((LANES,), jnp.float32)
        sems = (sem, sem2, sem3)

        pend = {0: fire_round(uv16, iv16, 0, sems),
                1: fire_round(uv16, iv16, 1, sems),
                2: fire_round(uv16, iv16, 2, sems)}
        for rw in range(RW):
            for c in pend.pop(rw):
                c.wait()
            if rw + 3 < RW:
                pend[rw + 3] = fire_round(uv16, iv16, rw + 3, sems)
            acc = compute_round(uv16, iv16, rw, acc)

        out_v[row, sl] = acc + ubch[row, sl] + ibch[row, sl]
        return carry

    lax.fori_loop(0, NGRP, group_body, 0)

    pltpu.sync_copy(out_v, out_hbm.at[wid])


@jax.jit
def _mf_sc(uidx, iidx, pt, qt, ub, ib):
    mesh = plsc.VectorSubcoreMesh(core_axis_name="c", subcore_axis_name="s")
    f = pl.kernel(
        _mf_body,
        out_type=jax.ShapeDtypeStruct((NW, NCHUNK, CHUNK), jnp.float32),
        mesh=mesh,
        compiler_params=pltpu.CompilerParams(
            needs_layout_passes=False, use_tc_tiling_on_sc=True),
        scratch_types=[
            pltpu.VMEM((NCHUNK, CHUNK), jnp.int32),
            pltpu.VMEM((NCHUNK, CHUNK), jnp.int32),
            pltpu.VMEM((12, 4, 8, 128), jnp.float32),
            pltpu.VMEM((12, 4, 8, 128), jnp.float32),
            pltpu.VMEM((NCHUNK, CHUNK), jnp.float32),
            pltpu.VMEM((NCHUNK, CHUNK), jnp.float32),
            pltpu.VMEM((NCHUNK, CHUNK), jnp.float32),
            pltpu.SemaphoreType.DMA,
            pltpu.SemaphoreType.DMA,
            pltpu.SemaphoreType.DMA,
        ],
    )
    return f(uidx, iidx, pt, qt, ub, ib)


def kernel(user_id, item_id, p, q, user_bias, item_bias):
    uidx = (user_id - 1).reshape(NW, NCHUNK, CHUNK)
    iidx = (item_id - 1).reshape(NW, NCHUNK, CHUNK)
    pt = jnp.transpose(p).reshape(4, 8, 1000000)
    qt = jnp.transpose(q).reshape(4, 8, 1000000)
    ub = jnp.sum(user_bias, axis=1)
    ib = jnp.sum(item_bias, axis=1)
    out = _mf_sc(uidx, iidx, pt, qt, ub, ib)
    return out.reshape(BATCH, 1)
